# baseline (speedup 1.0000x reference)
import functools

import jax
import jax.numpy as jnp
from jax import lax
from jax.experimental import pallas as pl
from jax.experimental.pallas import tpu as pltpu
from jax.experimental.pallas import tpu_sc as plsc

B = 16384
D = 32
L = 16
NC = 2
NS = 16
NW = NC * NS
BPW = B // NW
CH = 128
NCH = BPW // CH

_mesh = plsc.VectorSubcoreMesh(
    core_axis_name="c", subcore_axis_name="s", num_cores=NC, num_subcores=NS
)


@functools.partial(
    pl.kernel,
    out_type=jax.ShapeDtypeStruct((B // CH, CH), jnp.float32),
    mesh=_mesh,
    compiler_params=pltpu.CompilerParams(
        needs_layout_passes=False, use_tc_tiling_on_sc=False
    ),
    scratch_types=[
        pltpu.VMEM((NCH, CH), jnp.int32),
        pltpu.VMEM((NCH, CH), jnp.int32),
        pltpu.VMEM((D, NCH, CH), jnp.float32),
        pltpu.VMEM((D, NCH, CH), jnp.float32),
        pltpu.VMEM((NCH, CH), jnp.float32),
        pltpu.VMEM((NCH, CH), jnp.float32),
        pltpu.VMEM((NCH, CH), jnp.float32),
        pltpu.SemaphoreType.DMA,
    ],
)
def _mf_sc(sess_hbm, aids_hbm, semb_hbm, aemb_hbm, sbias_hbm, abias_hbm,
           out_hbm, sidx_v, aidx_v, scol_v, acol_v, sb_v, ab_v, out_v, sem):
    wid = lax.axis_index("s") * NC + lax.axis_index("c")
    base = wid * NCH

    pltpu.sync_copy(sess_hbm.at[pl.ds(base, NCH)], sidx_v)
    pltpu.sync_copy(aids_hbm.at[pl.ds(base, NCH)], aidx_v)

    copies = []
    for j in range(NCH):
        copies.append(pltpu.async_copy(sbias_hbm.at[0].at[sidx_v.at[j]], sb_v.at[j], sem))
        copies.append(pltpu.async_copy(abias_hbm.at[0].at[aidx_v.at[j]], ab_v.at[j], sem))
        for d in range(D):
            copies.append(
                pltpu.async_copy(semb_hbm.at[d].at[sidx_v.at[j]], scol_v.at[d, j], sem))
            copies.append(
                pltpu.async_copy(aemb_hbm.at[d].at[aidx_v.at[j]], acol_v.at[d, j], sem))
    for c in copies:
        c.wait()

    for jj in range(NCH):
        @pl.loop(0, CH // L)
        def _block(b):
            sl = pl.ds(b * L, L)
            acc = sb_v[jj, sl] + ab_v[jj, sl]
            for d in range(D):
                acc = acc + scol_v[d, jj, sl] * acol_v[d, jj, sl]
            out_v[jj, sl] = acc

    pltpu.sync_copy(out_v, out_hbm.at[pl.ds(base, NCH)])


def kernel(sessions, aids, session_emb, aid_emb, session_bias, aid_bias):
    sess = sessions.reshape(B // CH, CH).astype(jnp.int32)
    aid = aids.reshape(B // CH, CH).astype(jnp.int32)
    out = _mf_sc(
        sess,
        aid,
        session_emb.T,
        aid_emb.T,
        session_bias.T,
        aid_bias.T,
    )
    return out.reshape(B)


# speedup vs baseline: 1.0331x; 1.0331x over previous
import functools

import jax
import jax.numpy as jnp
from jax import lax
from jax.experimental import pallas as pl
from jax.experimental.pallas import tpu as pltpu
from jax.experimental.pallas import tpu_sc as plsc

B = 16384
D = 32
V = 1000000
L = 16
NC = 2
NS = 16
NW = NC * NS
BPW = B // NW
CH = 128
NCH = BPW // CH

_mesh = plsc.VectorSubcoreMesh(
    core_axis_name="c", subcore_axis_name="s", num_cores=NC, num_subcores=NS
)


@functools.partial(
    pl.kernel,
    out_type=jax.ShapeDtypeStruct((B // CH, CH), jnp.float32),
    mesh=_mesh,
    compiler_params=pltpu.CompilerParams(
        needs_layout_passes=False, use_tc_tiling_on_sc=False
    ),
    scratch_types=[
        pltpu.VMEM((NCH, CH), jnp.int32),
        pltpu.VMEM((NCH, CH), jnp.int32),
        pltpu.VMEM((D, NCH, CH), jnp.int32),
        pltpu.VMEM((D, NCH, CH), jnp.int32),
        pltpu.VMEM((D, NCH, CH), jnp.float32),
        pltpu.VMEM((D, NCH, CH), jnp.float32),
        pltpu.VMEM((NCH, CH), jnp.float32),
        pltpu.VMEM((NCH, CH), jnp.float32),
        pltpu.VMEM((NCH, CH), jnp.float32),
        pltpu.SemaphoreType.DMA,
    ],
)
def _mf_sc(sess_hbm, aids_hbm, semb_hbm, aemb_hbm, sbias_hbm, abias_hbm,
           out_hbm, sidx_v, aidx_v, sfi_v, afi_v, scol_v, acol_v,
           sb_v, ab_v, out_v, sem):
    wid = lax.axis_index("s") * NC + lax.axis_index("c")
    base = wid * NCH

    pltpu.sync_copy(sess_hbm.at[pl.ds(base, NCH)], sidx_v)
    pltpu.sync_copy(aids_hbm.at[pl.ds(base, NCH)], aidx_v)

    @pl.loop(0, D)
    def _build_d(d):
        off = d * V

        @pl.loop(0, NCH)
        def _build_j(j):
            @pl.loop(0, CH // L)
            def _build_v(v):
                sl = pl.ds(v * L, L)
                sfi_v[d, j, sl] = sidx_v[j, sl] + off
                afi_v[d, j, sl] = aidx_v[j, sl] + off

    copies = []
    for j in range(NCH):
        copies.append(pltpu.async_copy(sbias_hbm.at[sidx_v.at[j]], sb_v.at[j], sem))
        copies.append(pltpu.async_copy(abias_hbm.at[aidx_v.at[j]], ab_v.at[j], sem))
        for d in range(D):
            copies.append(
                pltpu.async_copy(semb_hbm.at[sfi_v.at[d].at[j]], scol_v.at[d, j], sem))
            copies.append(
                pltpu.async_copy(aemb_hbm.at[afi_v.at[d].at[j]], acol_v.at[d, j], sem))
    for c in copies:
        c.wait()

    for jj in range(NCH):
        @pl.loop(0, CH // L)
        def _block(b):
            sl = pl.ds(b * L, L)
            acc = sb_v[jj, sl] + ab_v[jj, sl]
            for d in range(D):
                acc = acc + scol_v[d, jj, sl] * acol_v[d, jj, sl]
            out_v[jj, sl] = acc

    pltpu.sync_copy(out_v, out_hbm.at[pl.ds(base, NCH)])


def kernel(sessions, aids, session_emb, aid_emb, session_bias, aid_bias):
    sess = sessions.reshape(B // CH, CH).astype(jnp.int32)
    aid = aids.reshape(B // CH, CH).astype(jnp.int32)
    out = _mf_sc(
        sess,
        aid,
        session_emb.T.reshape(-1),
        aid_emb.T.reshape(-1),
        session_bias.reshape(-1),
        aid_bias.reshape(-1),
    )
    return out.reshape(B)


# speedup vs baseline: 1.0350x; 1.0018x over previous
import functools

import jax
import jax.numpy as jnp
from jax import lax
from jax.experimental import pallas as pl
from jax.experimental.pallas import tpu as pltpu
from jax.experimental.pallas import tpu_sc as plsc

B = 16384
D = 32
V = 1000000
L = 16
NC = 2
NS = 16
NW = NC * NS
BPW = B // NW
NV = BPW // L

_mesh = plsc.VectorSubcoreMesh(
    core_axis_name="c", subcore_axis_name="s", num_cores=NC, num_subcores=NS
)


@functools.partial(
    pl.kernel,
    out_type=jax.ShapeDtypeStruct((B,), jnp.float32),
    mesh=_mesh,
    compiler_params=pltpu.CompilerParams(
        needs_layout_passes=False, use_tc_tiling_on_sc=False
    ),
    scratch_types=[
        pltpu.VMEM((BPW,), jnp.int32),
        pltpu.VMEM((BPW,), jnp.int32),
        pltpu.VMEM((D * BPW,), jnp.float32),
        pltpu.VMEM((D * BPW,), jnp.float32),
        pltpu.VMEM((BPW,), jnp.float32),
        pltpu.VMEM((BPW,), jnp.float32),
        pltpu.VMEM((BPW,), jnp.float32),
        pltpu.SemaphoreType.DMA,
    ],
)
def _mf_sc(sess_hbm, aids_hbm, semb_hbm, aemb_hbm, sbias_hbm, abias_hbm,
           out_hbm, sidx_v, aidx_v, scol_v, acol_v, sb_v, ab_v, out_v, sem):
    wid = lax.axis_index("s") * NC + lax.axis_index("c")
    base = wid * BPW

    pltpu.sync_copy(sess_hbm.at[pl.ds(base, BPW)], sidx_v)
    pltpu.sync_copy(aids_hbm.at[pl.ds(base, BPW)], aidx_v)

    @pl.loop(0, NV)
    def _bias(v):
        sl = pl.ds(v * L, L)
        pltpu.async_copy(sbias_hbm.at[sidx_v[sl]], sb_v.at[sl], sem)
        pltpu.async_copy(abias_hbm.at[aidx_v[sl]], ab_v.at[sl], sem)

    @pl.loop(0, D)
    def _gather_d(d):
        off = d * V

        @pl.loop(0, NV)
        def _gather_v(v):
            sl = pl.ds(v * L, L)
            dst = pl.ds(d * BPW + v * L, L)
            pltpu.async_copy(semb_hbm.at[sidx_v[sl] + off], scol_v.at[dst], sem)
            pltpu.async_copy(aemb_hbm.at[aidx_v[sl] + off], acol_v.at[dst], sem)

    pltpu.make_async_copy(semb_hbm.at[pl.ds(0, D * BPW)], scol_v, sem).wait()
    pltpu.make_async_copy(aemb_hbm.at[pl.ds(0, D * BPW)], acol_v, sem).wait()
    pltpu.make_async_copy(sbias_hbm.at[pl.ds(0, BPW)], sb_v, sem).wait()
    pltpu.make_async_copy(abias_hbm.at[pl.ds(0, BPW)], ab_v, sem).wait()

    @pl.loop(0, NV)
    def _block(b):
        sl = pl.ds(b * L, L)
        acc = sb_v[sl] + ab_v[sl]
        for d in range(D):
            dsl = pl.ds(d * BPW + b * L, L)
            acc = acc + scol_v[dsl] * acol_v[dsl]
        out_v[sl] = acc

    pltpu.sync_copy(out_v, out_hbm.at[pl.ds(base, BPW)])


def kernel(sessions, aids, session_emb, aid_emb, session_bias, aid_bias):
    sess = sessions.reshape(-1).astype(jnp.int32)
    aid = aids.reshape(-1).astype(jnp.int32)
    return _mf_sc(
        sess,
        aid,
        session_emb.T.reshape(-1),
        aid_emb.T.reshape(-1),
        session_bias.reshape(-1),
        aid_bias.reshape(-1),
    )


# speedup vs baseline: 5.9156x; 5.7154x over previous
import functools

import jax
import jax.numpy as jnp
from jax import lax
from jax.experimental import pallas as pl
from jax.experimental.pallas import tpu as pltpu
from jax.experimental.pallas import tpu_sc as plsc

B = 16384
D = 32
L = 16
NC = 2
NS = 16
NW = NC * NS
BPW = B // NW
CH = 128
NCH = BPW // CH

_mesh = plsc.VectorSubcoreMesh(
    core_axis_name="c", subcore_axis_name="s", num_cores=NC, num_subcores=NS
)


@functools.partial(
    pl.kernel,
    out_type=jax.ShapeDtypeStruct((B,), jnp.float32),
    mesh=_mesh,
    compiler_params=pltpu.CompilerParams(
        needs_layout_passes=False, use_tc_tiling_on_sc=False
    ),
    scratch_types=[
        pltpu.VMEM((NCH, CH), jnp.int32),
        pltpu.VMEM((NCH, CH), jnp.int32),
        pltpu.VMEM((BPW, D), jnp.float32),
        pltpu.VMEM((BPW, D), jnp.float32),
        pltpu.VMEM((BPW,), jnp.float32),
        pltpu.VMEM((BPW,), jnp.float32),
        pltpu.VMEM((BPW,), jnp.float32),
        pltpu.SemaphoreType.DMA,
    ],
)
def _mf_sc(sess_hbm, aids_hbm, semb_hbm, aemb_hbm, sbias_hbm, abias_hbm,
           out_hbm, sidx_v, aidx_v, srows_v, arows_v, sb_v, ab_v, out_v, sem):
    wid = lax.axis_index("s") * NC + lax.axis_index("c")

    pltpu.sync_copy(sess_hbm.at[pl.ds(wid * NCH, NCH)], sidx_v)
    pltpu.sync_copy(aids_hbm.at[pl.ds(wid * NCH, NCH)], aidx_v)

    copies = []
    for j in range(NCH):
        dst = pl.ds(j * CH, CH)
        copies.append(pltpu.async_copy(semb_hbm.at[sidx_v.at[j]], srows_v.at[dst], sem))
        copies.append(pltpu.async_copy(aemb_hbm.at[aidx_v.at[j]], arows_v.at[dst], sem))
        copies.append(pltpu.async_copy(sbias_hbm.at[sidx_v.at[j]], sb_v.at[dst], sem))
        copies.append(pltpu.async_copy(abias_hbm.at[aidx_v.at[j]], ab_v.at[dst], sem))
    for c in copies:
        c.wait()

    iota = lax.iota(jnp.int32, L)

    @pl.loop(0, BPW // L)
    def _block(b):
        row0 = b * L
        rows = row0 + iota
        acc = sb_v[pl.ds(row0, L)] + ab_v[pl.ds(row0, L)]
        for d in range(D):
            col = jnp.full((L,), d, jnp.int32)
            acc = acc + (plsc.load_gather(srows_v, [rows, col])
                         * plsc.load_gather(arows_v, [rows, col]))
        out_v[pl.ds(row0, L)] = acc

    pltpu.sync_copy(out_v, out_hbm.at[pl.ds(wid * BPW, BPW)])


def kernel(sessions, aids, session_emb, aid_emb, session_bias, aid_bias):
    sess = sessions.reshape(B // CH, CH).astype(jnp.int32)
    aid = aids.reshape(B // CH, CH).astype(jnp.int32)
    return _mf_sc(sess, aid, session_emb, aid_emb,
                  session_bias.reshape(-1), aid_bias.reshape(-1))
